# baseline (speedup 1.0000x reference)
import jax
import jax.numpy as jnp
from jax.experimental import pallas as pl
from jax.experimental.pallas import tpu as pltpu


def _reduce_scores(s):
    N, K = s.shape
    iota8 = jax.lax.broadcasted_iota(jnp.int32, (8, K), 0)
    run_m = s[0:8]
    run_i = iota8
    for i in range(1, N // 8):
        cur = s[8 * i:8 * i + 8]
        pred = cur > run_m
        run_m = jnp.where(pred, cur, run_m)
        run_i = jnp.where(pred, iota8 + jnp.int32(8 * i), run_i)
    m = jnp.max(run_m, axis=0, keepdims=True)
    idx = jnp.min(jnp.where(run_m == m, run_i, jnp.int32(N)),
                  axis=0, keepdims=True)
    return 1.0 - m, idx


def _proto_topk_kernel(x_ref, pt_ref, dist_ref, idx_ref, ptn_ref, sa_ref, sb_ref):
    g = pl.program_id(0)

    @pl.when(g == 0)
    def _normalize():
        pt = pt_ref[...]
        nrm = jnp.sqrt(jnp.sum(pt * pt, axis=0, keepdims=True))
        ptn_ref[...] = pt * (1.0 / jnp.maximum(nrm, 1e-12))

    def _step(mm_ref, rd_ref):
        mm_ref[...] = jax.lax.dot_general(
            x_ref[0], ptn_ref[...], (((1,), (0,)), ((), ())),
            preferred_element_type=jnp.float32,
        )
        dist, idx = _reduce_scores(rd_ref[...])
        dist_ref[0] = dist
        idx_ref[0] = idx

    @pl.when(g % 2 == 0)
    def _even():
        _step(sa_ref, sb_ref)

    @pl.when(g % 2 == 1)
    def _odd():
        _step(sb_ref, sa_ref)


def kernel(x, prototypes):
    B, N, D = x.shape
    K = prototypes.shape[0]
    pt = prototypes.T
    dist, idx = pl.pallas_call(
        _proto_topk_kernel,
        grid=(B + 1,),
        in_specs=[
            pl.BlockSpec((1, N, D), lambda g: (jnp.minimum(g, B - 1), 0, 0)),
            pl.BlockSpec((D, K), lambda g: (0, 0)),
        ],
        out_specs=[
            pl.BlockSpec((1, 1, K), lambda g: (jnp.maximum(g - 1, 0), 0, 0)),
            pl.BlockSpec((1, 1, K), lambda g: (jnp.maximum(g - 1, 0), 0, 0)),
        ],
        out_shape=[
            jax.ShapeDtypeStruct((B, 1, K), jnp.float32),
            jax.ShapeDtypeStruct((B, 1, K), jnp.int32),
        ],
        scratch_shapes=[
            pltpu.VMEM((D, K), jnp.float32),
            pltpu.VMEM((N, K), jnp.float32),
            pltpu.VMEM((N, K), jnp.float32),
        ],
    )(x, pt)
    return dist[:, 0, :], idx[:, 0, :]


# speedup vs baseline: 1.0967x; 1.0967x over previous
import jax
import jax.numpy as jnp
from jax.experimental import pallas as pl
from jax.experimental.pallas import tpu as pltpu


def _proto_topk_kernel(x_ref, pt_ref, dist_ref, idx_ref, ptn_ref):
    g = pl.program_id(0)

    @pl.when(g == 0)
    def _normalize():
        pt = pt_ref[...]
        nrm = jnp.sqrt(jnp.sum(pt * pt, axis=0, keepdims=True))
        ptn_ref[...] = pt * (1.0 / jnp.maximum(nrm, 1e-12))

    s = jax.lax.dot_general(
        x_ref[0], ptn_ref[...], (((1,), (0,)), ((), ())),
        preferred_element_type=jnp.float32,
    )
    N, K = s.shape
    iota8 = jax.lax.broadcasted_iota(jnp.int32, (8, K), 0)
    run_m = s[0:8]
    run_i = iota8
    for i in range(1, N // 8):
        cur = s[8 * i:8 * i + 8]
        pred = cur > run_m
        run_m = jnp.where(pred, cur, run_m)
        run_i = jnp.where(pred, iota8 + jnp.int32(8 * i), run_i)
    m = jnp.max(run_m, axis=0, keepdims=True)
    idx = jnp.min(jnp.where(run_m == m, run_i, jnp.int32(N)),
                  axis=0, keepdims=True)
    dist_ref[0] = 1.0 - m
    idx_ref[0] = idx


def kernel(x, prototypes):
    B, N, D = x.shape
    K = prototypes.shape[0]
    pt = prototypes.T
    dist, idx = pl.pallas_call(
        _proto_topk_kernel,
        grid=(B,),
        in_specs=[
            pl.BlockSpec((1, N, D), lambda g: (g, 0, 0)),
            pl.BlockSpec((D, K), lambda g: (0, 0)),
        ],
        out_specs=[
            pl.BlockSpec((1, 1, K), lambda g: (g, 0, 0)),
            pl.BlockSpec((1, 1, K), lambda g: (g, 0, 0)),
        ],
        out_shape=[
            jax.ShapeDtypeStruct((B, 1, K), jnp.float32),
            jax.ShapeDtypeStruct((B, 1, K), jnp.int32),
        ],
        scratch_shapes=[pltpu.VMEM((D, K), jnp.float32)],
    )(x, pt)
    return dist[:, 0, :], idx[:, 0, :]


# speedup vs baseline: 2.0939x; 1.9093x over previous
import jax
import jax.numpy as jnp
from jax.experimental import pallas as pl
from jax.experimental.pallas import tpu as pltpu


_BB = 8


def _reduce_scores(s):
    N, K = s.shape
    iota8 = jax.lax.broadcasted_iota(jnp.int32, (8, K), 0)
    run_m = s[0:8]
    run_i = iota8
    for i in range(1, N // 8):
        cur = s[8 * i:8 * i + 8]
        pred = cur > run_m
        run_m = jnp.where(pred, cur, run_m)
        run_i = jnp.where(pred, iota8 + jnp.int32(8 * i), run_i)
    m = jnp.max(run_m, axis=0, keepdims=True)
    idx = jnp.min(jnp.where(run_m == m, run_i, jnp.int32(N)),
                  axis=0, keepdims=True)
    return 1.0 - m, idx


def _proto_topk_kernel(x_ref, pt_ref, dist_ref, idx_ref, ptn_ref):
    g = pl.program_id(0)

    @pl.when(g == 0)
    def _normalize():
        pt = pt_ref[...]
        nrm = jnp.sqrt(jnp.sum(pt * pt, axis=0, keepdims=True))
        ptn_ref[...] = pt * (1.0 / jnp.maximum(nrm, 1e-12))

    for j in range(_BB):
        s = jax.lax.dot_general(
            x_ref[j], ptn_ref[...], (((1,), (0,)), ((), ())),
            preferred_element_type=jnp.float32,
        )
        dist, idx = _reduce_scores(s)
        dist_ref[j] = dist
        idx_ref[j] = idx


def kernel(x, prototypes):
    B, N, D = x.shape
    K = prototypes.shape[0]
    pt = prototypes.T
    dist, idx = pl.pallas_call(
        _proto_topk_kernel,
        grid=(B // _BB,),
        in_specs=[
            pl.BlockSpec((_BB, N, D), lambda g: (g, 0, 0)),
            pl.BlockSpec((D, K), lambda g: (0, 0)),
        ],
        out_specs=[
            pl.BlockSpec((_BB, 1, K), lambda g: (g, 0, 0)),
            pl.BlockSpec((_BB, 1, K), lambda g: (g, 0, 0)),
        ],
        out_shape=[
            jax.ShapeDtypeStruct((B, 1, K), jnp.float32),
            jax.ShapeDtypeStruct((B, 1, K), jnp.int32),
        ],
        scratch_shapes=[pltpu.VMEM((D, K), jnp.float32)],
    )(x, pt)
    return dist[:, 0, :], idx[:, 0, :]


# speedup vs baseline: 2.3416x; 1.1183x over previous
import jax
import jax.numpy as jnp
from jax.experimental import pallas as pl
from jax.experimental.pallas import tpu as pltpu


_BB = 8


def _reduce_scores(s):
    N, K = s.shape
    iota8 = jax.lax.broadcasted_iota(jnp.int32, (8, K), 0)
    run_m = s[0:8]
    run_i = iota8
    for i in range(1, N // 8):
        cur = s[8 * i:8 * i + 8]
        pred = cur > run_m
        run_m = jnp.where(pred, cur, run_m)
        run_i = jnp.where(pred, iota8 + jnp.int32(8 * i), run_i)
    m = jnp.max(run_m, axis=0, keepdims=True)
    idx = jnp.min(jnp.where(run_m == m, run_i, jnp.int32(N)),
                  axis=0, keepdims=True)
    return 1.0 - m, idx


def _proto_topk_kernel(x_ref, pt_ref, dist_ref, idx_ref, ptn_ref):
    g = pl.program_id(0)

    @pl.when(g == 0)
    def _normalize():
        pt = pt_ref[...]
        nrm = jnp.sqrt(jnp.sum(pt * pt, axis=0, keepdims=True))
        ptn_ref[...] = pt * (1.0 / jnp.maximum(nrm, 1e-12))

    for j in range(_BB):
        s = jax.lax.dot_general(
            x_ref[j], ptn_ref[...], (((1,), (0,)), ((), ())),
            preferred_element_type=jnp.float32,
        )
        dist, idx = _reduce_scores(s)
        dist_ref[j:j + 1] = dist
        idx_ref[j:j + 1] = idx


def kernel(x, prototypes):
    B, N, D = x.shape
    K = prototypes.shape[0]
    pt = prototypes.T
    dist, idx = pl.pallas_call(
        _proto_topk_kernel,
        grid=(B // _BB,),
        in_specs=[
            pl.BlockSpec((_BB, N, D), lambda g: (g, 0, 0)),
            pl.BlockSpec((D, K), lambda g: (0, 0)),
        ],
        out_specs=[
            pl.BlockSpec((_BB, K), lambda g: (g, 0)),
            pl.BlockSpec((_BB, K), lambda g: (g, 0)),
        ],
        out_shape=[
            jax.ShapeDtypeStruct((B, K), jnp.float32),
            jax.ShapeDtypeStruct((B, K), jnp.int32),
        ],
        scratch_shapes=[pltpu.VMEM((D, K), jnp.float32)],
    )(x, pt)
    return dist, idx


# speedup vs baseline: 2.5464x; 1.0874x over previous
import jax
import jax.numpy as jnp
from jax.experimental import pallas as pl
from jax.experimental.pallas import tpu as pltpu


_BB = 8


def _reduce_scores(s):
    N, K = s.shape
    iota8 = jax.lax.broadcasted_iota(jnp.int32, (8, K), 0)
    run_m = s[0:8]
    run_i = iota8
    for i in range(1, N // 8):
        cur = s[8 * i:8 * i + 8]
        pred = cur > run_m
        run_m = jnp.where(pred, cur, run_m)
        run_i = jnp.where(pred, iota8 + jnp.int32(8 * i), run_i)
    m = jnp.max(run_m, axis=0, keepdims=True)
    idx = jnp.min(jnp.where(run_m == m, run_i, jnp.int32(N)),
                  axis=0, keepdims=True)
    return 1.0 - m, idx


def _proto_topk_kernel(x_ref, p_ref, dist_ref, idx_ref, ptn_ref):
    g = pl.program_id(0)

    @pl.when(g == 0)
    def _normalize():
        pt = jnp.transpose(p_ref[...])
        nrm = jnp.sqrt(jnp.sum(pt * pt, axis=0, keepdims=True))
        ptn_ref[...] = pt * (1.0 / jnp.maximum(nrm, 1e-12))

    for j in range(_BB):
        s = jax.lax.dot_general(
            x_ref[j], ptn_ref[...], (((1,), (0,)), ((), ())),
            preferred_element_type=jnp.float32,
        )
        dist, idx = _reduce_scores(s)
        dist_ref[j:j + 1] = dist
        idx_ref[j:j + 1] = idx


def kernel(x, prototypes):
    B, N, D = x.shape
    K = prototypes.shape[0]
    dist, idx = pl.pallas_call(
        _proto_topk_kernel,
        grid=(B // _BB,),
        in_specs=[
            pl.BlockSpec((_BB, N, D), lambda g: (g, 0, 0)),
            pl.BlockSpec((K, D), lambda g: (0, 0)),
        ],
        out_specs=[
            pl.BlockSpec((_BB, K), lambda g: (g, 0)),
            pl.BlockSpec((_BB, K), lambda g: (g, 0)),
        ],
        out_shape=[
            jax.ShapeDtypeStruct((B, K), jnp.float32),
            jax.ShapeDtypeStruct((B, K), jnp.int32),
        ],
        scratch_shapes=[pltpu.VMEM((D, K), jnp.float32)],
    )(x, prototypes)
    return dist, idx
